# Initial kernel scaffold; baseline (speedup 1.0000x reference)
import jax
import jax.numpy as jnp
from jax.experimental import pallas as pl


def kernel(samples, enc_weight, enc_bias, class_weight):
    raise NotImplementedError("write your pallas kernel here")


# speedup vs baseline: 2.1871x; 2.1871x over previous
import functools

import jax
import jax.numpy as jnp
from jax.experimental import pallas as pl
from jax.experimental.pallas import tpu as pltpu


def _flowhd_kernel(s_ref, ew_ref, bias_ref, cw_ref, out_ref,
                   sim_acc, qn2_acc, wn2_acc, *, n_chan):
    d = pl.program_id(1)
    nd = pl.num_programs(1)

    @pl.when(d == 0)
    def _init():
        sim_acc[...] = jnp.zeros_like(sim_acc)
        qn2_acc[...] = jnp.zeros_like(qn2_acc)
        wn2_acc[...] = jnp.zeros_like(wn2_acc)

    ew = ew_ref[...]
    bias = bias_ref[...]

    acc = jnp.zeros((s_ref.shape[0], ew.shape[0]), dtype=jnp.float32)
    for c in range(n_chan):
        x = s_ref[:, c, :]
        proj = jax.lax.dot_general(
            x, ew, (((1,), (1,)), ((), ())),
            preferred_element_type=jnp.float32)
        acc = acc + jnp.sin(2.0 * proj + bias)
    summed = 0.5 * acc - (0.5 * n_chan) * jnp.sin(bias)
    q = jnp.tanh(summed)

    qn2_acc[...] += jnp.sum(q * q, axis=1, keepdims=True)

    cw = cw_ref[...]
    sim_acc[...] += jax.lax.dot_general(
        q, cw, (((1,), (1,)), ((), ())),
        preferred_element_type=jnp.float32)
    wn2_acc[...] += jnp.sum(cw * cw, axis=1, keepdims=True).reshape(1, -1)

    @pl.when(d == nd - 1)
    def _finish():
        qn = jnp.sqrt(qn2_acc[...])
        wn = jnp.sqrt(wn2_acc[...])
        out_ref[...] = sim_acc[...] / (qn * wn + 1e-12)


@jax.jit
def kernel(samples, enc_weight, enc_bias, class_weight):
    B, C, F = samples.shape
    D = enc_weight.shape[0]
    K = class_weight.shape[0]

    T = 1024
    Bt = 256
    Dpad = ((D + T - 1) // T) * T
    pad = Dpad - D
    ew = jnp.pad(enc_weight, ((0, pad), (0, 0)))
    bias = jnp.pad(enc_bias, ((0, 0), (0, pad)))
    cw = jnp.pad(class_weight, ((0, 0), (0, pad)))

    grid = (B // Bt, Dpad // T)
    return pl.pallas_call(
        functools.partial(_flowhd_kernel, n_chan=C),
        grid=grid,
        in_specs=[
            pl.BlockSpec((Bt, C, F), lambda b, d: (b, 0, 0)),
            pl.BlockSpec((T, F), lambda b, d: (d, 0)),
            pl.BlockSpec((1, T), lambda b, d: (0, d)),
            pl.BlockSpec((K, T), lambda b, d: (0, d)),
        ],
        out_specs=pl.BlockSpec((Bt, K), lambda b, d: (b, 0)),
        out_shape=jax.ShapeDtypeStruct((B, K), jnp.float32),
        scratch_shapes=[
            pltpu.VMEM((Bt, K), jnp.float32),
            pltpu.VMEM((Bt, 1), jnp.float32),
            pltpu.VMEM((1, K), jnp.float32),
        ],
        compiler_params=pltpu.CompilerParams(
            dimension_semantics=("arbitrary", "arbitrary")),
    )(samples, ew, bias, cw)


# speedup vs baseline: 9.1048x; 4.1629x over previous
import functools

import jax
import jax.numpy as jnp
from jax.experimental import pallas as pl
from jax.experimental.pallas import tpu as pltpu


def _fast_sin(x):
    r = x * 0.15915494309189535
    r = r - jnp.round(r)
    r2 = r * r
    p = jnp.float32(-12.372395737099913)
    p = p * r2 + jnp.float32(41.26987033307637)
    p = p * r2 + jnp.float32(-76.59491552319069)
    p = p * r2 + jnp.float32(81.597656706991)
    p = p * r2 + jnp.float32(-41.34148031326184)
    p = p * r2 + jnp.float32(6.283183465946359)
    return p * r


def _flowhd_kernel(s_ref, ew_ref, bias_ref, cw_ref, out_ref,
                   sim_acc, qn2_acc, wn2_acc, *, n_chan):
    d = pl.program_id(1)
    nd = pl.num_programs(1)

    @pl.when(d == 0)
    def _init():
        sim_acc[...] = jnp.zeros_like(sim_acc)
        qn2_acc[...] = jnp.zeros_like(qn2_acc)
        wn2_acc[...] = jnp.zeros_like(wn2_acc)

    ew = ew_ref[...]
    bias = bias_ref[...]

    acc = jnp.zeros((s_ref.shape[0], ew.shape[0]), dtype=jnp.float32)
    for c in range(n_chan):
        x = s_ref[:, c, :]
        proj = jax.lax.dot_general(
            x, ew, (((1,), (1,)), ((), ())),
            preferred_element_type=jnp.float32)
        acc = acc + _fast_sin(2.0 * proj + bias)
    summed = 0.5 * acc - (0.5 * n_chan) * _fast_sin(bias)
    q = jnp.tanh(summed)

    qn2_acc[...] += jnp.sum(q * q, axis=1, keepdims=True)

    cw = cw_ref[...]
    sim_acc[...] += jax.lax.dot_general(
        q, cw, (((1,), (1,)), ((), ())),
        preferred_element_type=jnp.float32)
    wn2_acc[...] += jnp.sum(cw * cw, axis=1, keepdims=True).reshape(1, -1)

    @pl.when(d == nd - 1)
    def _finish():
        qn = jnp.sqrt(qn2_acc[...])
        wn = jnp.sqrt(wn2_acc[...])
        out_ref[...] = sim_acc[...] / (qn * wn + 1e-12)


@jax.jit
def kernel(samples, enc_weight, enc_bias, class_weight):
    B, C, F = samples.shape
    D = enc_weight.shape[0]
    K = class_weight.shape[0]

    T = 1024
    Bt = 256
    Dpad = ((D + T - 1) // T) * T
    pad = Dpad - D
    ew = jnp.pad(enc_weight, ((0, pad), (0, 0)))
    bias = jnp.pad(enc_bias, ((0, 0), (0, pad)))
    cw = jnp.pad(class_weight, ((0, 0), (0, pad)))

    grid = (B // Bt, Dpad // T)
    return pl.pallas_call(
        functools.partial(_flowhd_kernel, n_chan=C),
        grid=grid,
        in_specs=[
            pl.BlockSpec((Bt, C, F), lambda b, d: (b, 0, 0)),
            pl.BlockSpec((T, F), lambda b, d: (d, 0)),
            pl.BlockSpec((1, T), lambda b, d: (0, d)),
            pl.BlockSpec((K, T), lambda b, d: (0, d)),
        ],
        out_specs=pl.BlockSpec((Bt, K), lambda b, d: (b, 0)),
        out_shape=jax.ShapeDtypeStruct((B, K), jnp.float32),
        scratch_shapes=[
            pltpu.VMEM((Bt, K), jnp.float32),
            pltpu.VMEM((Bt, 1), jnp.float32),
            pltpu.VMEM((1, K), jnp.float32),
        ],
        compiler_params=pltpu.CompilerParams(
            dimension_semantics=("parallel", "arbitrary")),
    )(samples, ew, bias, cw)


# speedup vs baseline: 11.3201x; 1.2433x over previous
import functools

import jax
import jax.numpy as jnp
from jax.experimental import pallas as pl
from jax.experimental.pallas import tpu as pltpu

_C1 = 3.1398652231383195
_C3 = -20.568124594883363
_C5 = 39.16349904954592
_C7 = -28.557916107728236


def _half_sin_wrapped(t):
    r = t - jnp.round(t)
    r2 = r * r
    p = jnp.float32(_C7)
    p = p * r2 + jnp.float32(_C5)
    p = p * r2 + jnp.float32(_C3)
    p = p * r2 + jnp.float32(_C1)
    return p * r


def _flowhd_kernel(s_ref, ew_ref, bias_ref, cw_ref, out_ref,
                   sim_acc, qn2_acc, wn2_acc, *, n_chan):
    d = pl.program_id(1)
    nd = pl.num_programs(1)

    @pl.when(d == 0)
    def _init():
        sim_acc[...] = jnp.zeros_like(sim_acc)
        qn2_acc[...] = jnp.zeros_like(qn2_acc)
        wn2_acc[...] = jnp.zeros_like(wn2_acc)

    ew = ew_ref[...]
    bias = bias_ref[...]

    acc = jnp.zeros((s_ref.shape[0], ew.shape[0]), dtype=jnp.float32)
    for c in range(n_chan):
        x = s_ref[:, c, :]
        proj = jax.lax.dot_general(
            x, ew, (((1,), (1,)), ((), ())),
            preferred_element_type=jnp.float32)
        acc = acc + _half_sin_wrapped(proj + bias)
    summed = acc - n_chan * _half_sin_wrapped(bias)
    q = jnp.tanh(summed)

    ones_col = jnp.ones((q.shape[1], 1), dtype=jnp.float32)
    qn2_acc[...] += jax.lax.dot_general(
        q * q, ones_col, (((1,), (0,)), ((), ())),
        preferred_element_type=jnp.float32)

    cw = cw_ref[...]
    sim_acc[...] += jax.lax.dot_general(
        q, cw, (((1,), (1,)), ((), ())),
        preferred_element_type=jnp.float32)
    ones_row = jnp.ones((1, cw.shape[1]), dtype=jnp.float32)
    wn2_acc[...] += jax.lax.dot_general(
        ones_row, cw * cw, (((1,), (1,)), ((), ())),
        preferred_element_type=jnp.float32)

    @pl.when(d == nd - 1)
    def _finish():
        qn = jnp.sqrt(qn2_acc[...])
        wn = jnp.sqrt(wn2_acc[...])
        out_ref[...] = sim_acc[...] / (qn * wn + 1e-12)


@jax.jit
def kernel(samples, enc_weight, enc_bias, class_weight):
    B, C, F = samples.shape
    D = enc_weight.shape[0]
    K = class_weight.shape[0]

    T = 1024
    Bt = 256
    Dpad = ((D + T - 1) // T) * T
    pad = Dpad - D
    inv_pi = 0.3183098861837907
    ew = jnp.pad(enc_weight * inv_pi, ((0, pad), (0, 0)))
    bias = jnp.pad(enc_bias * (0.5 * inv_pi), ((0, 0), (0, pad)))
    cw = jnp.pad(class_weight, ((0, 0), (0, pad)))

    grid = (B // Bt, Dpad // T)
    return pl.pallas_call(
        functools.partial(_flowhd_kernel, n_chan=C),
        grid=grid,
        in_specs=[
            pl.BlockSpec((Bt, C, F), lambda b, d: (b, 0, 0)),
            pl.BlockSpec((T, F), lambda b, d: (d, 0)),
            pl.BlockSpec((1, T), lambda b, d: (0, d)),
            pl.BlockSpec((K, T), lambda b, d: (0, d)),
        ],
        out_specs=pl.BlockSpec((Bt, K), lambda b, d: (b, 0)),
        out_shape=jax.ShapeDtypeStruct((B, K), jnp.float32),
        scratch_shapes=[
            pltpu.VMEM((Bt, K), jnp.float32),
            pltpu.VMEM((Bt, 1), jnp.float32),
            pltpu.VMEM((1, K), jnp.float32),
        ],
        compiler_params=pltpu.CompilerParams(
            dimension_semantics=("parallel", "arbitrary")),
    )(samples, ew, bias, cw)
